# baseline (speedup 1.0000x reference)
import functools

import jax
import jax.numpy as jnp
from jax import lax
from jax.experimental import pallas as pl
from jax.experimental.pallas import tpu as pltpu
from jax.experimental.pallas import tpu_sc as plsc

_N = 268
_E = _N * _N
_B = 32
_EP = 72192
_CW = _EP // 4


_PG = 8


def _prep_body(ei1_ref, ea1_ref, ei2_ref, ea2_ref,
               i1_ref, w1_ref, i2_ref, w2_ref):
    zi = jnp.zeros((_PG, _EP - _E), jnp.int32)
    zf = jnp.zeros((_PG, _EP - _E), jnp.float32)
    for ei_ref, ea_ref, i_ref, w_ref in (
            (ei1_ref, ea1_ref, i1_ref, w1_ref),
            (ei2_ref, ea2_ref, i2_ref, w2_ref)):
        r = ei_ref[:, 0, :]
        c = ei_ref[:, 1, :]
        i_ref[:, pl.ds(0, _E)] = r * _N + c
        i_ref[:, pl.ds(_E, _EP - _E)] = zi
        w_ref[:, pl.ds(0, _E)] = ea_ref[...]
        w_ref[:, pl.ds(_E, _EP - _E)] = zf


def _prep_stage(ei1, ea1, ei2, ea2):
    return pl.pallas_call(
        _prep_body,
        grid=(_B // _PG,),
        in_specs=[
            pl.BlockSpec((_PG, 2, _E), lambda g: (g, 0, 0)),
            pl.BlockSpec((_PG, _E), lambda g: (g, 0)),
            pl.BlockSpec((_PG, 2, _E), lambda g: (g, 0, 0)),
            pl.BlockSpec((_PG, _E), lambda g: (g, 0)),
        ],
        out_specs=[pl.BlockSpec((_PG, _EP), lambda g: (g, 0))] * 4,
        out_shape=[
            jax.ShapeDtypeStruct((_B, _EP), jnp.int32),
            jax.ShapeDtypeStruct((_B, _EP), jnp.float32),
            jax.ShapeDtypeStruct((_B, _EP), jnp.int32),
            jax.ShapeDtypeStruct((_B, _EP), jnp.float32),
        ],
    )(ei1, ea1, ei2, ea2)


def _densify_body(idx_hbm, w_hbm, out_hbm, idx_v, w_v, zero_v, shared):
    cid = lax.axis_index("c")
    sid = lax.axis_index("s")
    g = sid * 2 + cid
    base = pl.multiple_of(sid * _EP, 128)
    acc = shared.at[pl.ds(base, _EP)]

    def zfill(i, carry):
        zero_v[pl.ds(i * 16, 16)] = jnp.zeros((16,), jnp.float32)
        return carry

    lax.fori_loop(0, _CW // 16, zfill, 0)

    for chunk in range(4):
        pltpu.sync_copy(zero_v, shared.at[pl.ds(base + chunk * _CW, _CW)])
    for chunk in range(4):
        pltpu.sync_copy(idx_hbm.at[g, pl.ds(chunk * _CW, _CW)], idx_v)
        pltpu.sync_copy(w_hbm.at[g, pl.ds(chunk * _CW, _CW)], w_v)
        pltpu.sync_copy(w_v, acc.at[idx_v], add=True)
    pltpu.sync_copy(acc, out_hbm.at[g])


@functools.cache
def _densify():
    return pl.kernel(
        _densify_body,
        out_type=jax.ShapeDtypeStruct((_B, _EP), jnp.float32),
        mesh=plsc.VectorSubcoreMesh(core_axis_name="c", subcore_axis_name="s"),
        scratch_types=[
            pltpu.VMEM((_CW,), jnp.int32),
            pltpu.VMEM((_CW,), jnp.float32),
            pltpu.VMEM((_CW,), jnp.float32),
            pltpu.VMEM_SHARED((16 * _EP,), jnp.float32),
        ],
    )


def _dense_body(a_ref, x_ref, w1_ref, b1_ref, w4_ref,
                b4_ref, wc1_ref, bc1_ref, wc2_ref, bc2_ref, wc3_ref, bc3_ref,
                o_ref):
    f32 = jnp.float32
    cn = (((0,), (0,)), ((), ()))

    def run(a, x):
        r = lax.broadcasted_iota(jnp.int32, (_N, _N), 0)
        c = lax.broadcasted_iota(jnp.int32, (_N, _N), 1)
        a = jnp.where(r == c, 0.0, a)
        deg = jnp.sum(a, axis=1, keepdims=True)
        dis = jnp.where(deg > 0, lax.rsqrt(jnp.where(deg > 0, deg, 1.0)),
                        0.0)

        def matvec(h):
            y = lax.dot_general(a, h * dis, cn, preferred_element_type=f32)
            return -dis * y

        def conv(h, w_ref, b_ref):
            t1 = matvec(h)
            t2 = 2.0 * matvec(t1) - h
            out = (jnp.dot(h, w_ref[0], preferred_element_type=f32)
                   + jnp.dot(t1, w_ref[1], preferred_element_type=f32)
                   + jnp.dot(t2, w_ref[2], preferred_element_type=f32)
                   + b_ref[...])
            return jnp.maximum(out, 0.0)

        h = conv(x, w1_ref, b1_ref)
        h = conv(h, w4_ref, b4_ref)
        z = lax.dot_general(wc1_ref[...], h, cn, preferred_element_type=f32)
        z = jnp.maximum(z + bc1_ref[...], 0.0)
        z = lax.dot_general(wc2_ref[...], z, cn, preferred_element_type=f32)
        z = jnp.maximum(z + bc2_ref[...], 0.0)
        o = lax.dot_general(wc3_ref[...], z, cn, preferred_element_type=f32)
        return o + bc3_ref[...]

    o_ref[0] = run(a_ref[0], x_ref[0])


def _dense_stage(a, x, w1, b1, w4, b4, wc1, bc1, wc2, bc2, wc3, bc3):
    nc = w4.shape[2]
    full = lambda shp: pl.BlockSpec(shp, lambda g: (0,) * len(shp))
    return pl.pallas_call(
        _dense_body,
        grid=(_B,),
        in_specs=[
            pl.BlockSpec((1, _N, _N), lambda g: (g, 0, 0)),
            pl.BlockSpec((1, _N, _N), lambda g: (g, 0, 0)),
            full(w1.shape), full(b1.shape), full(w4.shape), full(b4.shape),
            full(wc1.shape), full(bc1.shape), full(wc2.shape),
            full(bc2.shape), full(wc3.shape), full(bc3.shape),
        ],
        out_specs=[pl.BlockSpec((1, 1, nc), lambda g: (g, 0, 0))],
        out_shape=[jax.ShapeDtypeStruct((_B, 1, nc), jnp.float32)],
    )(a, x, w1, b1, w4, b4, wc1, bc1, wc2, bc2, wc3, bc3)


def kernel(x1, edge_index1, edge_attr1, x2, edge_index2, edge_attr2,
           W_gc1, b_gc1, W_gc4, b_gc4, Wc1, bc1, Wc2, bc2, Wc3, bc3):
    i1, w1, i2, w2 = _prep_stage(edge_index1, edge_attr1,
                                 edge_index2, edge_attr2)

    densify = _densify()
    d1 = densify(i1, w1)
    d2 = densify(i2, w2)
    a1 = d1[:, :_E].reshape(_B, _N, _N)
    a2 = d2[:, :_E].reshape(_B, _N, _N)

    nc = W_gc4.shape[2]
    wargs = (W_gc1, b_gc1.reshape(1, -1), W_gc4, b_gc4.reshape(1, -1),
             Wc1, bc1.reshape(-1, 1), Wc2, bc2.reshape(-1, 1), Wc3,
             bc3.reshape(1, 1))
    (o1,) = _dense_stage(a1, x1, *wargs)
    (o2,) = _dense_stage(a2, x2, *wargs)
    return o1.reshape(_B, nc, 1), o2.reshape(_B, nc, 1)


# speedup vs baseline: 1.0454x; 1.0454x over previous
import functools

import jax
import jax.numpy as jnp
from jax import lax
from jax.experimental import pallas as pl
from jax.experimental.pallas import tpu as pltpu
from jax.experimental.pallas import tpu_sc as plsc

_N = 268
_E = _N * _N
_B = 32
_EP = 72192
_CW = _EP // 4


_PG = 8


def _prep_body(ei1_ref, ea1_ref, ei2_ref, ea2_ref,
               i1_ref, w1_ref, i2_ref, w2_ref):
    zi = jnp.zeros((_PG, _EP - _E), jnp.int32)
    zf = jnp.zeros((_PG, _EP - _E), jnp.float32)
    for ei_ref, ea_ref, i_ref, w_ref in (
            (ei1_ref, ea1_ref, i1_ref, w1_ref),
            (ei2_ref, ea2_ref, i2_ref, w2_ref)):
        r = ei_ref[:, 0, :]
        c = ei_ref[:, 1, :]
        i_ref[:, pl.ds(0, _E)] = r * _N + c
        i_ref[:, pl.ds(_E, _EP - _E)] = zi
        w_ref[:, pl.ds(0, _E)] = ea_ref[...]
        w_ref[:, pl.ds(_E, _EP - _E)] = zf


def _prep_stage(ei1, ea1, ei2, ea2):
    return pl.pallas_call(
        _prep_body,
        grid=(_B // _PG,),
        in_specs=[
            pl.BlockSpec((_PG, 2, _E), lambda g: (g, 0, 0)),
            pl.BlockSpec((_PG, _E), lambda g: (g, 0)),
            pl.BlockSpec((_PG, 2, _E), lambda g: (g, 0, 0)),
            pl.BlockSpec((_PG, _E), lambda g: (g, 0)),
        ],
        out_specs=[pl.BlockSpec((_PG, _EP), lambda g: (g, 0))] * 4,
        out_shape=[
            jax.ShapeDtypeStruct((_B, _EP), jnp.int32),
            jax.ShapeDtypeStruct((_B, _EP), jnp.float32),
            jax.ShapeDtypeStruct((_B, _EP), jnp.int32),
            jax.ShapeDtypeStruct((_B, _EP), jnp.float32),
        ],
    )(ei1, ea1, ei2, ea2)


def _densify_body(idx1_hbm, w1_hbm, idx2_hbm, w2_hbm, out_hbm,
                  idx_v, w_v, zero_v, shared):
    cid = lax.axis_index("c")
    sid = lax.axis_index("s")
    g = sid * 2 + cid
    base = pl.multiple_of(sid * _EP, 128)
    acc = shared.at[pl.ds(base, _EP)]

    def zfill(i, carry):
        zero_v[pl.ds(i * 16, 16)] = jnp.zeros((16,), jnp.float32)
        return carry

    lax.fori_loop(0, _CW // 16, zfill, 0)

    for side, (ih, wh) in enumerate(((idx1_hbm, w1_hbm),
                                     (idx2_hbm, w2_hbm))):
        for chunk in range(4):
            pltpu.sync_copy(zero_v, shared.at[pl.ds(base + chunk * _CW, _CW)])
        for chunk in range(4):
            pltpu.sync_copy(ih.at[g, pl.ds(chunk * _CW, _CW)], idx_v)
            pltpu.sync_copy(wh.at[g, pl.ds(chunk * _CW, _CW)], w_v)
            pltpu.sync_copy(w_v, acc.at[idx_v], add=True)
        pltpu.sync_copy(acc, out_hbm.at[side, g])


@functools.cache
def _densify():
    return pl.kernel(
        _densify_body,
        out_type=jax.ShapeDtypeStruct((2, _B, _EP), jnp.float32),
        mesh=plsc.VectorSubcoreMesh(core_axis_name="c", subcore_axis_name="s"),
        scratch_types=[
            pltpu.VMEM((_CW,), jnp.int32),
            pltpu.VMEM((_CW,), jnp.float32),
            pltpu.VMEM((_CW,), jnp.float32),
            pltpu.VMEM_SHARED((16 * _EP,), jnp.float32),
        ],
    )


def _dense_body(a1_ref, a2_ref, x1_ref, x2_ref, w1_ref, b1_ref, w4_ref,
                b4_ref, wc1_ref, bc1_ref, wc2_ref, bc2_ref, wc3_ref, bc3_ref,
                o1_ref, o2_ref):
    f32 = jnp.float32
    cn = (((0,), (0,)), ((), ()))

    def run(a, x):
        r = lax.broadcasted_iota(jnp.int32, (_N, _N), 0)
        c = lax.broadcasted_iota(jnp.int32, (_N, _N), 1)
        a = jnp.where(r == c, 0.0, a)
        deg = jnp.sum(a, axis=1, keepdims=True)
        dis = jnp.where(deg > 0, lax.rsqrt(jnp.where(deg > 0, deg, 1.0)),
                        0.0)

        def matvec(h):
            y = lax.dot_general(a, h * dis, cn, preferred_element_type=f32)
            return -dis * y

        def conv(h, w_ref, b_ref):
            t1 = matvec(h)
            t2 = 2.0 * matvec(t1) - h
            out = (jnp.dot(h, w_ref[0], preferred_element_type=f32)
                   + jnp.dot(t1, w_ref[1], preferred_element_type=f32)
                   + jnp.dot(t2, w_ref[2], preferred_element_type=f32)
                   + b_ref[...])
            return jnp.maximum(out, 0.0)

        h = conv(x, w1_ref, b1_ref)
        h = conv(h, w4_ref, b4_ref)
        z = lax.dot_general(wc1_ref[...], h, cn, preferred_element_type=f32)
        z = jnp.maximum(z + bc1_ref[...], 0.0)
        z = lax.dot_general(wc2_ref[...], z, cn, preferred_element_type=f32)
        z = jnp.maximum(z + bc2_ref[...], 0.0)
        o = lax.dot_general(wc3_ref[...], z, cn, preferred_element_type=f32)
        return o + bc3_ref[...]

    o1_ref[0] = run(a1_ref[0, 0], x1_ref[0])
    o2_ref[0] = run(a2_ref[0, 0], x2_ref[0])


def _dense_stage(a_all, x1, x2, w1, b1, w4, b4, wc1, bc1, wc2, bc2, wc3, bc3):
    nh = w1.shape[2]
    nc = w4.shape[2]
    full = lambda shp: pl.BlockSpec(shp, lambda g: (0,) * len(shp))
    return pl.pallas_call(
        _dense_body,
        grid=(_B,),
        in_specs=[
            pl.BlockSpec((1, 1, _N, _N), lambda g: (0, g, 0, 0)),
            pl.BlockSpec((1, 1, _N, _N), lambda g: (1, g, 0, 0)),
            pl.BlockSpec((1, _N, _N), lambda g: (g, 0, 0)),
            pl.BlockSpec((1, _N, _N), lambda g: (g, 0, 0)),
            full(w1.shape), full(b1.shape), full(w4.shape), full(b4.shape),
            full(wc1.shape), full(bc1.shape), full(wc2.shape),
            full(bc2.shape), full(wc3.shape), full(bc3.shape),
        ],
        out_specs=[
            pl.BlockSpec((1, 1, nc), lambda g: (g, 0, 0)),
            pl.BlockSpec((1, 1, nc), lambda g: (g, 0, 0)),
        ],
        out_shape=[
            jax.ShapeDtypeStruct((_B, 1, nc), jnp.float32),
            jax.ShapeDtypeStruct((_B, 1, nc), jnp.float32),
        ],
    )(a_all, a_all, x1, x2, w1, b1, w4, b4, wc1, bc1, wc2, bc2, wc3, bc3)


def kernel(x1, edge_index1, edge_attr1, x2, edge_index2, edge_attr2,
           W_gc1, b_gc1, W_gc4, b_gc4, Wc1, bc1, Wc2, bc2, Wc3, bc3):
    i1, w1, i2, w2 = _prep_stage(edge_index1, edge_attr1,
                                 edge_index2, edge_attr2)

    dense = _densify()(i1, w1, i2, w2)
    a_all = dense[:, :, :_E].reshape(2, _B, _N, _N)

    nc = W_gc4.shape[2]
    o1, o2 = _dense_stage(
        a_all, x1, x2, W_gc1, b_gc1.reshape(1, -1), W_gc4,
        b_gc4.reshape(1, -1), Wc1, bc1.reshape(-1, 1), Wc2,
        bc2.reshape(-1, 1), Wc3, bc3.reshape(1, 1))
    return o1.reshape(_B, nc, 1), o2.reshape(_B, nc, 1)
